# baseline (speedup 1.0000x reference)
import functools

import jax
import jax.numpy as jnp
from jax import lax
from jax.experimental import pallas as pl
from jax.experimental.pallas import tpu as pltpu
from jax.experimental.pallas import tpu_sc as plsc

NC = 2
NS = 16
NW = NC * NS

_CHUNK = 125


def _sc_agg(table, src4, dst4, z_feat, z_deg, *, with_deg):
    n, d = table.shape
    ngroup = src4.shape[1]
    rows_a = ((n + NS - 1) // NS + 7) // 8 * 8
    rows_last = n - rows_a * (NS - 1)
    mesh = plsc.VectorSubcoreMesh(
        core_axis_name="c", subcore_axis_name="s", num_cores=NC,
        num_subcores=NS)

    out_type = [jax.ShapeDtypeStruct((NC, n, d), jnp.float32)]
    scratch = [
        pltpu.VMEM((2, 2, _CHUNK), jnp.int32),
        pltpu.VMEM((2, 2, _CHUNK), jnp.int32),
        pltpu.VMEM((2, _CHUNK, d), jnp.float32),
        [pltpu.SemaphoreType.DMA] * 2,
        [pltpu.SemaphoreType.DMA] * 2,
        [pltpu.SemaphoreType.DMA] * 2,
        [pltpu.SemaphoreType.DMA] * 2,
        pltpu.VMEM_SHARED((n, d), jnp.float32),
    ]
    if with_deg:
        out_type.append(jax.ShapeDtypeStruct((NC, n, 16), jnp.float32))
        scratch += [
            pltpu.VMEM((_CHUNK, 16), jnp.float32),
            pltpu.VMEM_SHARED((n, 16), jnp.float32),
        ]

    @functools.partial(pl.kernel, out_type=out_type, mesh=mesh,
                       scratch_types=scratch,
                       compiler_params=pltpu.CompilerParams(
                           use_tc_tiling_on_sc=False))
    def k(table_hbm, src_hbm, dst_hbm, zf_hbm, zd_hbm, *refs):
        if with_deg:
            (agg_out, deg_out, sidx, didx, rows_v, gsem, ssem, isem, dsem,
             agg_sh, ones_v, deg_sh) = refs
        else:
            (agg_out, sidx, didx, rows_v, gsem, ssem, isem, dsem,
             agg_sh) = refs
        cid = lax.axis_index("c")
        sid = lax.axis_index("s")
        wid = cid * NS + sid

        pltpu.sync_copy(dst_hbm.at[wid, 0], didx.at[0])
        pltpu.async_copy(dst_hbm.at[wid, 1], didx.at[1], dsem[1])
        pltpu.sync_copy(src_hbm.at[wid, 0], sidx.at[0])
        pltpu.async_copy(src_hbm.at[wid, 1], sidx.at[1], isem[1])
        row0 = sid * rows_a

        def zero_slice(nrows):
            sl = pl.ds(row0, nrows)
            pltpu.sync_copy(zf_hbm.at[sl], agg_sh.at[sl])
            if with_deg:
                pltpu.sync_copy(zd_hbm.at[sl], deg_sh.at[sl])

        pl.when(sid < NS - 1)(lambda: zero_slice(rows_a))
        pl.when(sid == NS - 1)(lambda: zero_slice(rows_last))
        if with_deg:

            @pl.loop(0, _CHUNK)
            def _(r):
                ones_v[r, :] = jnp.ones((16,), jnp.float32)

        for b in range(2):
            pltpu.async_copy(table_hbm.at[sidx.at[0, b]], rows_v.at[b],
                             gsem[b])

        plsc.subcore_barrier()

        def step(g, G, H, b):
            pltpu.make_async_copy(table_hbm.at[sidx.at[0, b]],
                                  rows_v.at[b], gsem[b]).wait()
            if b == 0:
                @pl.when(g < ngroup - 1)
                def _():
                    pltpu.make_async_copy(src_hbm.at[wid, 0], sidx.at[H],
                                          isem[H]).wait()
                    pltpu.make_async_copy(dst_hbm.at[wid, 0], didx.at[H],
                                          dsem[H]).wait()
            pltpu.async_copy(rows_v.at[b], agg_sh.at[didx.at[G, b]],
                             ssem[b], add=True)
            if with_deg:
                pltpu.sync_copy(ones_v, deg_sh.at[didx.at[G, b]], add=True)
            if b == 1:
                @pl.when(g < ngroup - 2)
                def _():
                    pltpu.async_copy(src_hbm.at[wid, g + 2], sidx.at[G],
                                     isem[G])

            def drain_scatter():
                pltpu.make_async_copy(rows_v.at[b],
                                      agg_sh.at[didx.at[G, b]],
                                      ssem[b]).wait()

            @pl.when(g < ngroup - 1)
            def _():
                drain_scatter()
                pltpu.async_copy(table_hbm.at[sidx.at[H, b]],
                                 rows_v.at[b], gsem[b])
                if b == 1:
                    @pl.when(g < ngroup - 2)
                    def _():
                        pltpu.async_copy(dst_hbm.at[wid, g + 2],
                                         didx.at[G], dsem[G])

            pl.when(g == ngroup - 1)(drain_scatter)

        @pl.loop(0, ngroup // 2)
        def _(t):
            for b in range(2):
                step(2 * t, 0, 1, b)
            for b in range(2):
                step(2 * t + 1, 1, 0, b)

        plsc.subcore_barrier()

        def copy_out(nrows):
            sl = pl.ds(row0, nrows)
            pltpu.sync_copy(agg_sh.at[sl], agg_out.at[cid, sl])
            if with_deg:
                pltpu.sync_copy(deg_sh.at[sl], deg_out.at[cid, sl])

        pl.when(sid < NS - 1)(lambda: copy_out(rows_a))
        pl.when(sid == NS - 1)(lambda: copy_out(rows_last))

    return k(table, src4, dst4, z_feat, z_deg)


def _tc_self(x, w_self, b):
    n, d = x.shape
    r = 1000

    def body(x_ref, ws_ref, b_ref, o_ref):
        o_ref[...] = jnp.dot(x_ref[...], ws_ref[...],
                             preferred_element_type=jnp.float32) + b_ref[...]

    return pl.pallas_call(
        body,
        grid=(n // r,),
        in_specs=[
            pl.BlockSpec((r, d), lambda i: (i, 0)),
            pl.BlockSpec((d, d), lambda i: (0, 0)),
            pl.BlockSpec((1, d), lambda i: (0, 0)),
        ],
        out_specs=pl.BlockSpec((r, d), lambda i: (i, 0)),
        out_shape=jax.ShapeDtypeStruct((n, d), jnp.float32),
    )(x, w_self, b.reshape(1, d))


def _tc_combine(s, agg, deg, w_neigh, *, relu):
    n, d = s.shape
    r = 1000

    def body(s_ref, agg_ref, deg_ref, wn_ref, o_ref):
        degs = deg_ref[0] + deg_ref[1]
        inv = 1.0 / jnp.maximum(degs[:, :1], 1.0)
        mean = (agg_ref[0] + agg_ref[1]) * inv
        out = s_ref[...] + jnp.dot(mean, wn_ref[...],
                                   preferred_element_type=jnp.float32)
        o_ref[...] = jnp.maximum(out, 0.0) if relu else out

    return pl.pallas_call(
        body,
        grid=(n // r,),
        in_specs=[
            pl.BlockSpec((r, d), lambda i: (i, 0)),
            pl.BlockSpec((NC, r, d), lambda i: (0, i, 0)),
            pl.BlockSpec((NC, r, 16), lambda i: (0, i, 0)),
            pl.BlockSpec((d, d), lambda i: (0, 0)),
        ],
        out_specs=pl.BlockSpec((r, d), lambda i: (i, 0)),
        out_shape=jax.ShapeDtypeStruct((n, d), jnp.float32),
    )(s, agg, deg, w_neigh)


def _tc_combine_self(s, agg, deg, w_neigh, w_self2, b2):
    n, d = s.shape
    r = 1000

    def body(s_ref, agg_ref, deg_ref, wn_ref, ws2_ref, b2_ref, h_ref,
             s2_ref):
        degs = deg_ref[0] + deg_ref[1]
        inv = 1.0 / jnp.maximum(degs[:, :1], 1.0)
        mean = (agg_ref[0] + agg_ref[1]) * inv
        h = jnp.maximum(
            s_ref[...] + jnp.dot(mean, wn_ref[...],
                                 preferred_element_type=jnp.float32), 0.0)
        h_ref[...] = h
        s2_ref[...] = jnp.dot(h, ws2_ref[...],
                              preferred_element_type=jnp.float32) + b2_ref[...]

    return pl.pallas_call(
        body,
        grid=(n // r,),
        in_specs=[
            pl.BlockSpec((r, d), lambda i: (i, 0)),
            pl.BlockSpec((NC, r, d), lambda i: (0, i, 0)),
            pl.BlockSpec((NC, r, 16), lambda i: (0, i, 0)),
            pl.BlockSpec((d, d), lambda i: (0, 0)),
            pl.BlockSpec((d, d), lambda i: (0, 0)),
            pl.BlockSpec((1, d), lambda i: (0, 0)),
        ],
        out_specs=[
            pl.BlockSpec((r, d), lambda i: (i, 0)),
            pl.BlockSpec((r, d), lambda i: (i, 0)),
        ],
        out_shape=[
            jax.ShapeDtypeStruct((n, d), jnp.float32),
            jax.ShapeDtypeStruct((n, d), jnp.float32),
        ],
    )(s, agg, deg, w_neigh, w_self2, b2.reshape(1, d))


def kernel(x, edge_index, W1_self, W1_neigh, b1, W2_self, W2_neigh, b2):
    n, d = x.shape
    e = edge_index.shape[1]
    e_per_w = e // NW
    nchunk = e_per_w // _CHUNK
    src4 = edge_index[0].reshape(NW, nchunk // 2, 2, _CHUNK)
    dst4 = edge_index[1].reshape(NW, nchunk // 2, 2, _CHUNK)
    z_feat = jnp.zeros((n, d), jnp.float32)
    z_deg = jnp.zeros((n, 16), jnp.float32)

    agg1, deg = _sc_agg(x, src4, dst4, z_feat, z_deg, with_deg=True)
    s1 = _tc_self(x, W1_self, b1)
    h, s2 = _tc_combine_self(s1, agg1, deg, W1_neigh, W2_self, b2)
    (agg2,) = _sc_agg(h, src4, dst4, z_feat, z_deg, with_deg=False)
    out = _tc_combine(s2, agg2, deg, W2_neigh, relu=False)
    return out


# speedup vs baseline: 1.0144x; 1.0144x over previous
import functools

import jax
import jax.numpy as jnp
from jax import lax
from jax.experimental import pallas as pl
from jax.experimental.pallas import tpu as pltpu
from jax.experimental.pallas import tpu_sc as plsc

NC = 2
NS = 16
NW = NC * NS

_CHUNK = 125


def _sc_agg(table, src4, dst4, z_feat, z_deg, *, with_deg):
    n, d = table.shape
    ngroup = src4.shape[1]
    rows_a = ((n + NS - 1) // NS + 7) // 8 * 8
    rows_last = n - rows_a * (NS - 1)
    mesh = plsc.VectorSubcoreMesh(
        core_axis_name="c", subcore_axis_name="s", num_cores=NC,
        num_subcores=NS)

    out_type = [jax.ShapeDtypeStruct((NC, n, d), jnp.float32)]
    scratch = [
        pltpu.VMEM((2, 2, _CHUNK), jnp.int32),
        pltpu.VMEM((2, 2, _CHUNK), jnp.int32),
        pltpu.VMEM((2, _CHUNK, d), jnp.float32),
        [pltpu.SemaphoreType.DMA] * 2,
        [pltpu.SemaphoreType.DMA] * 2,
        [pltpu.SemaphoreType.DMA] * 2,
        [pltpu.SemaphoreType.DMA] * 2,
        pltpu.VMEM_SHARED((n, d), jnp.float32),
    ]
    if with_deg:
        out_type.append(jax.ShapeDtypeStruct((NC, n, 16), jnp.float32))
        scratch += [
            pltpu.VMEM((_CHUNK, 16), jnp.float32),
            pltpu.VMEM_SHARED((n, 16), jnp.float32),
        ]

    @functools.partial(pl.kernel, out_type=out_type, mesh=mesh,
                       scratch_types=scratch,
                       compiler_params=pltpu.CompilerParams(
                           use_tc_tiling_on_sc=False))
    def k(table_hbm, src_hbm, dst_hbm, zf_hbm, zd_hbm, *refs):
        if with_deg:
            (agg_out, deg_out, sidx, didx, rows_v, gsem, ssem, isem, dsem,
             agg_sh, ones_v, deg_sh) = refs
        else:
            (agg_out, sidx, didx, rows_v, gsem, ssem, isem, dsem,
             agg_sh) = refs
        cid = lax.axis_index("c")
        sid = lax.axis_index("s")
        wid = cid * NS + sid

        pltpu.sync_copy(src_hbm.at[wid, 0], sidx.at[0])
        for b in range(2):
            pltpu.async_copy(table_hbm.at[sidx.at[0, b]], rows_v.at[b],
                             gsem[b])
        pltpu.async_copy(src_hbm.at[wid, 1], sidx.at[1], isem[1])
        pltpu.async_copy(dst_hbm.at[wid, 0], didx.at[0], dsem[0])
        pltpu.async_copy(dst_hbm.at[wid, 1], didx.at[1], dsem[1])
        row0 = sid * rows_a

        def zero_slice(nrows):
            sl = pl.ds(row0, nrows)
            pltpu.sync_copy(zf_hbm.at[sl], agg_sh.at[sl])
            if with_deg:
                pltpu.sync_copy(zd_hbm.at[sl], deg_sh.at[sl])

        pl.when(sid < NS - 1)(lambda: zero_slice(rows_a))
        pl.when(sid == NS - 1)(lambda: zero_slice(rows_last))
        if with_deg:

            @pl.loop(0, _CHUNK)
            def _(r):
                ones_v[r, :] = jnp.ones((16,), jnp.float32)

        pltpu.make_async_copy(dst_hbm.at[wid, 0], didx.at[0],
                              dsem[0]).wait()
        plsc.subcore_barrier()

        def step(g, G, H, b):
            pltpu.make_async_copy(table_hbm.at[sidx.at[0, b]],
                                  rows_v.at[b], gsem[b]).wait()
            if b == 0:
                @pl.when(g < ngroup - 1)
                def _():
                    pltpu.make_async_copy(src_hbm.at[wid, 0], sidx.at[H],
                                          isem[H]).wait()
                    pltpu.make_async_copy(dst_hbm.at[wid, 0], didx.at[H],
                                          dsem[H]).wait()
            pltpu.async_copy(rows_v.at[b], agg_sh.at[didx.at[G, b]],
                             ssem[b], add=True)
            if with_deg:
                pltpu.sync_copy(ones_v, deg_sh.at[didx.at[G, b]], add=True)
            if b == 1:
                @pl.when(g < ngroup - 2)
                def _():
                    pltpu.async_copy(src_hbm.at[wid, g + 2], sidx.at[G],
                                     isem[G])

            def drain_scatter():
                pltpu.make_async_copy(rows_v.at[b],
                                      agg_sh.at[didx.at[G, b]],
                                      ssem[b]).wait()

            @pl.when(g < ngroup - 1)
            def _():
                drain_scatter()
                pltpu.async_copy(table_hbm.at[sidx.at[H, b]],
                                 rows_v.at[b], gsem[b])
                if b == 1:
                    @pl.when(g < ngroup - 2)
                    def _():
                        pltpu.async_copy(dst_hbm.at[wid, g + 2],
                                         didx.at[G], dsem[G])

            pl.when(g == ngroup - 1)(drain_scatter)

        @pl.loop(0, ngroup // 2)
        def _(t):
            for b in range(2):
                step(2 * t, 0, 1, b)
            for b in range(2):
                step(2 * t + 1, 1, 0, b)

        plsc.subcore_barrier()

        def copy_out(nrows):
            sl = pl.ds(row0, nrows)
            pltpu.sync_copy(agg_sh.at[sl], agg_out.at[cid, sl])
            if with_deg:
                pltpu.sync_copy(deg_sh.at[sl], deg_out.at[cid, sl])

        pl.when(sid < NS - 1)(lambda: copy_out(rows_a))
        pl.when(sid == NS - 1)(lambda: copy_out(rows_last))

    return k(table, src4, dst4, z_feat, z_deg)


def _tc_self(x, w_self, b):
    n, d = x.shape
    r = 1000

    def body(x_ref, ws_ref, b_ref, o_ref):
        o_ref[...] = jnp.dot(x_ref[...], ws_ref[...],
                             preferred_element_type=jnp.float32) + b_ref[...]

    return pl.pallas_call(
        body,
        grid=(n // r,),
        in_specs=[
            pl.BlockSpec((r, d), lambda i: (i, 0)),
            pl.BlockSpec((d, d), lambda i: (0, 0)),
            pl.BlockSpec((1, d), lambda i: (0, 0)),
        ],
        out_specs=pl.BlockSpec((r, d), lambda i: (i, 0)),
        out_shape=jax.ShapeDtypeStruct((n, d), jnp.float32),
    )(x, w_self, b.reshape(1, d))


def _tc_combine(s, agg, deg, w_neigh, *, relu):
    n, d = s.shape
    r = 1000

    def body(s_ref, agg_ref, deg_ref, wn_ref, o_ref):
        degs = deg_ref[0] + deg_ref[1]
        inv = 1.0 / jnp.maximum(degs[:, :1], 1.0)
        mean = (agg_ref[0] + agg_ref[1]) * inv
        out = s_ref[...] + jnp.dot(mean, wn_ref[...],
                                   preferred_element_type=jnp.float32)
        o_ref[...] = jnp.maximum(out, 0.0) if relu else out

    return pl.pallas_call(
        body,
        grid=(n // r,),
        in_specs=[
            pl.BlockSpec((r, d), lambda i: (i, 0)),
            pl.BlockSpec((NC, r, d), lambda i: (0, i, 0)),
            pl.BlockSpec((NC, r, 16), lambda i: (0, i, 0)),
            pl.BlockSpec((d, d), lambda i: (0, 0)),
        ],
        out_specs=pl.BlockSpec((r, d), lambda i: (i, 0)),
        out_shape=jax.ShapeDtypeStruct((n, d), jnp.float32),
    )(s, agg, deg, w_neigh)


def kernel(x, edge_index, W1_self, W1_neigh, b1, W2_self, W2_neigh, b2):
    n, d = x.shape
    e = edge_index.shape[1]
    e_per_w = e // NW
    nchunk = e_per_w // _CHUNK
    src4 = edge_index[0].reshape(NW, nchunk // 2, 2, _CHUNK)
    dst4 = edge_index[1].reshape(NW, nchunk // 2, 2, _CHUNK)
    z_feat = jnp.zeros((n, d), jnp.float32)
    z_deg = jnp.zeros((n, 16), jnp.float32)

    agg1, deg = _sc_agg(x, src4, dst4, z_feat, z_deg, with_deg=True)
    s1 = _tc_self(x, W1_self, b1)
    h = _tc_combine(s1, agg1, deg, W1_neigh, relu=True)
    (agg2,) = _sc_agg(h, src4, dst4, z_feat, z_deg, with_deg=False)
    s2 = _tc_self(h, W2_self, b2)
    out = _tc_combine(s2, agg2, deg, W2_neigh, relu=False)
    return out


# speedup vs baseline: 1.0781x; 1.0628x over previous
import functools

import jax
import jax.numpy as jnp
from jax import lax
from jax.experimental import pallas as pl
from jax.experimental.pallas import tpu as pltpu
from jax.experimental.pallas import tpu_sc as plsc

NC = 2
NS = 16
NW = NC * NS

_CHUNK = 100


def _sc_agg(table, src2, dst2, z_feat, z_deg, *, with_deg):
    n, d = table.shape
    nchunk = src2.shape[1]
    rows_a = ((n + NS - 1) // NS + 7) // 8 * 8
    rows_last = n - rows_a * (NS - 1)
    mesh = plsc.VectorSubcoreMesh(
        core_axis_name="c", subcore_axis_name="s", num_cores=NC,
        num_subcores=NS)

    out_type = [jax.ShapeDtypeStruct((NC, n, d), jnp.float32)]
    scratch = [
        pltpu.VMEM((3, _CHUNK), jnp.int32),
        pltpu.VMEM((3, _CHUNK), jnp.int32),
        pltpu.VMEM((3, _CHUNK, d), jnp.float32),
        [pltpu.SemaphoreType.DMA] * 3,
        [pltpu.SemaphoreType.DMA] * 3,
        [pltpu.SemaphoreType.DMA] * 3,
        [pltpu.SemaphoreType.DMA] * 3,
        pltpu.VMEM_SHARED((n, d), jnp.float32),
    ]
    if with_deg:
        out_type.append(jax.ShapeDtypeStruct((NC, n, 16), jnp.float32))
        scratch += [
            pltpu.VMEM((_CHUNK, 16), jnp.float32),
            pltpu.VMEM_SHARED((n, 16), jnp.float32),
        ]

    @functools.partial(pl.kernel, out_type=out_type, mesh=mesh,
                       scratch_types=scratch,
                       compiler_params=pltpu.CompilerParams(
                           use_tc_tiling_on_sc=False))
    def k(table_hbm, src_hbm, dst_hbm, zf_hbm, zd_hbm, *refs):
        if with_deg:
            (agg_out, deg_out, sidx, didx, rows_v, gsem, ssem, isem, dsem,
             agg_sh, ones_v, deg_sh) = refs
        else:
            (agg_out, sidx, didx, rows_v, gsem, ssem, isem, dsem,
             agg_sh) = refs
        cid = lax.axis_index("c")
        sid = lax.axis_index("s")
        wid = cid * NS + sid

        pltpu.sync_copy(src_hbm.at[wid, 0], sidx.at[0])
        pltpu.async_copy(table_hbm.at[sidx.at[0]], rows_v.at[0], gsem[0])
        pltpu.sync_copy(src_hbm.at[wid, 1], sidx.at[1])
        pltpu.async_copy(table_hbm.at[sidx.at[1]], rows_v.at[1], gsem[1])
        pltpu.async_copy(src_hbm.at[wid, 2], sidx.at[2], isem[2])
        pltpu.async_copy(dst_hbm.at[wid, 0], didx.at[0], dsem[0])
        pltpu.async_copy(dst_hbm.at[wid, 1], didx.at[1], dsem[1])
        row0 = sid * rows_a

        def zero_slice(nrows):
            sl = pl.ds(row0, nrows)
            pltpu.sync_copy(zf_hbm.at[sl], agg_sh.at[sl])
            if with_deg:
                pltpu.sync_copy(zd_hbm.at[sl], deg_sh.at[sl])

        pl.when(sid < NS - 1)(lambda: zero_slice(rows_a))
        pl.when(sid == NS - 1)(lambda: zero_slice(rows_last))
        if with_deg:

            @pl.loop(0, _CHUNK)
            def _(r):
                ones_v[r, :] = jnp.ones((16,), jnp.float32)

        plsc.subcore_barrier()

        def step(j, r, p):
            pltpu.make_async_copy(table_hbm.at[sidx.at[r]], rows_v.at[r],
                                  gsem[r]).wait()
            pltpu.make_async_copy(dst_hbm.at[wid, 0], didx.at[r],
                                  dsem[r]).wait()
            pltpu.async_copy(rows_v.at[r], agg_sh.at[didx.at[r]],
                             ssem[r], add=True)
            if with_deg:
                pltpu.sync_copy(ones_v, deg_sh.at[didx.at[r]], add=True)

            @pl.when(j + 3 < nchunk)
            def _():
                pltpu.async_copy(src_hbm.at[wid, j + 3], sidx.at[r],
                                 isem[r])

            @pl.when(j >= 1)
            def _():
                pltpu.make_async_copy(rows_v.at[p], agg_sh.at[didx.at[p]],
                                      ssem[p]).wait()

            @pl.when(j + 2 < nchunk)
            def _():
                pltpu.async_copy(dst_hbm.at[wid, j + 2], didx.at[p],
                                 dsem[p])
                pltpu.make_async_copy(src_hbm.at[wid, 0], sidx.at[p],
                                      isem[p]).wait()
                pltpu.async_copy(table_hbm.at[sidx.at[p]], rows_v.at[p],
                                 gsem[p])

        nfull = nchunk // 3

        @pl.loop(0, nfull)
        def _(t):
            step(3 * t, 0, 2)
            step(3 * t + 1, 1, 0)
            step(3 * t + 2, 2, 1)

        for j in range(3 * nfull, nchunk):
            step(jnp.int32(j), j % 3, (j + 2) % 3)

        last = (nchunk - 1) % 3
        pltpu.make_async_copy(rows_v.at[last],
                              agg_sh.at[didx.at[last]], ssem[last]).wait()

        plsc.subcore_barrier()

        def copy_out(nrows):
            sl = pl.ds(row0, nrows)
            pltpu.sync_copy(agg_sh.at[sl], agg_out.at[cid, sl])
            if with_deg:
                pltpu.sync_copy(deg_sh.at[sl], deg_out.at[cid, sl])

        pl.when(sid < NS - 1)(lambda: copy_out(rows_a))
        pl.when(sid == NS - 1)(lambda: copy_out(rows_last))

    return k(table, src2, dst2, z_feat, z_deg)


def _tc_self(x, w_self, b):
    n, d = x.shape
    r = 1000

    def body(x_ref, ws_ref, b_ref, o_ref):
        o_ref[...] = jnp.dot(x_ref[...], ws_ref[...],
                             preferred_element_type=jnp.float32) + b_ref[...]

    return pl.pallas_call(
        body,
        grid=(n // r,),
        in_specs=[
            pl.BlockSpec((r, d), lambda i: (i, 0)),
            pl.BlockSpec((d, d), lambda i: (0, 0)),
            pl.BlockSpec((1, d), lambda i: (0, 0)),
        ],
        out_specs=pl.BlockSpec((r, d), lambda i: (i, 0)),
        out_shape=jax.ShapeDtypeStruct((n, d), jnp.float32),
    )(x, w_self, b.reshape(1, d))


def _tc_combine(s, agg, deg, w_neigh, *, relu):
    n, d = s.shape
    r = 1000

    def body(s_ref, agg_ref, deg_ref, wn_ref, o_ref):
        degs = deg_ref[0] + deg_ref[1]
        inv = 1.0 / jnp.maximum(degs[:, :1], 1.0)
        mean = (agg_ref[0] + agg_ref[1]) * inv
        out = s_ref[...] + jnp.dot(mean, wn_ref[...],
                                   preferred_element_type=jnp.float32)
        o_ref[...] = jnp.maximum(out, 0.0) if relu else out

    return pl.pallas_call(
        body,
        grid=(n // r,),
        in_specs=[
            pl.BlockSpec((r, d), lambda i: (i, 0)),
            pl.BlockSpec((NC, r, d), lambda i: (0, i, 0)),
            pl.BlockSpec((NC, r, 16), lambda i: (0, i, 0)),
            pl.BlockSpec((d, d), lambda i: (0, 0)),
        ],
        out_specs=pl.BlockSpec((r, d), lambda i: (i, 0)),
        out_shape=jax.ShapeDtypeStruct((n, d), jnp.float32),
    )(s, agg, deg, w_neigh)


def kernel(x, edge_index, W1_self, W1_neigh, b1, W2_self, W2_neigh, b2):
    n, d = x.shape
    e = edge_index.shape[1]
    e_per_w = e // NW
    nchunk = e_per_w // _CHUNK
    src2 = edge_index[0].reshape(NW, nchunk, _CHUNK)
    dst2 = edge_index[1].reshape(NW, nchunk, _CHUNK)
    z_feat = jnp.zeros((n, d), jnp.float32)
    z_deg = jnp.zeros((n, 16), jnp.float32)

    agg1, deg = _sc_agg(x, src2, dst2, z_feat, z_deg, with_deg=True)
    s1 = _tc_self(x, W1_self, b1)
    h = _tc_combine(s1, agg1, deg, W1_neigh, relu=True)
    (agg2,) = _sc_agg(h, src2, dst2, z_feat, z_deg, with_deg=False)
    s2 = _tc_self(h, W2_self, b2)
    out = _tc_combine(s2, agg2, deg, W2_neigh, relu=False)
    return out
